# baseline (speedup 1.0000x reference)
import functools

import jax
import jax.numpy as jnp
from jax import lax
from jax.experimental import pallas as pl
from jax.experimental.pallas import tpu as pltpu
from jax.experimental.pallas import tpu_sc as plsc

HIDDEN = 64
IDX_MINOR = 128
SUBLANES = 8
G = 4
NWORKERS = 32


@jax.jit
def _sc_gather(ids4d, table):
    sblk, bblk = ids4d.shape[0], ids4d.shape[1]
    seq, batch = sblk * SUBLANES, bblk * IDX_MINOR
    n_chunks = seq // G

    mesh = plsc.VectorSubcoreMesh(core_axis_name="c", subcore_axis_name="s")

    @functools.partial(
        pl.kernel,
        mesh=mesh,
        out_type=jax.ShapeDtypeStruct((seq, batch, HIDDEN), jnp.float32),
        scratch_types=[
            pltpu.VMEM((G, IDX_MINOR), jnp.int32),
            pltpu.VMEM((G, IDX_MINOR), jnp.int32),
            pltpu.VMEM((G, IDX_MINOR, HIDDEN), jnp.float32),
            pltpu.VMEM((G, IDX_MINOR, HIDDEN), jnp.float32),
            pltpu.SemaphoreType.DMA,
            pltpu.SemaphoreType.DMA,
            pltpu.SemaphoreType.DMA,
            pltpu.SemaphoreType.DMA,
        ],
        compiler_params=pltpu.CompilerParams(use_tc_tiling_on_sc=False),
    )
    def k(ids_hbm, table_hbm, out_hbm, idx0, idx1, rows0, rows1,
          sg0, sg1, sw0, sw1):
        wid = lax.axis_index("s") * 2 + lax.axis_index("c")
        idx_v = (idx0, idx1)
        rows_v = (rows0, rows1)
        sem_g = (sg0, sg1)
        sem_w = (sw0, sw1)

        def load_and_gather(c, b):
            s_tile = c // (SUBLANES // G)
            s_half = c % (SUBLANES // G)
            pltpu.sync_copy(
                ids_hbm.at[s_tile, wid, pl.ds(s_half * G, G)],
                idx_v[b])
            for j in range(G):
                pltpu.async_copy(
                    table_hbm.at[idx_v[b].at[j]],
                    rows_v[b].at[j],
                    sem_g[b],
                )

        def gather_drain(b):
            pltpu.make_async_copy(
                out_hbm.at[pl.ds(0, G), pl.ds(0, IDX_MINOR)],
                rows_v[b], sem_g[b]).wait()

        def write_start(c, b):
            pltpu.async_copy(
                rows_v[b],
                out_hbm.at[pl.ds(c * G, G), pl.ds(wid * IDX_MINOR, IDX_MINOR)],
                sem_w[b])

        def write_drain(b):
            pltpu.make_async_copy(
                out_hbm.at[pl.ds(0, G), pl.ds(0, IDX_MINOR)],
                rows_v[b], sem_w[b]).wait()

        load_and_gather(0, 0)
        load_and_gather(1, 1)
        gather_drain(0)
        write_start(0, 0)

        def body(i, carry):
            for b in range(2):
                c = 2 * i + 2 + b
                write_drain(b)
                load_and_gather(c, b)
                gather_drain(1 - b)
                write_start(c - 1, 1 - b)
            return carry

        lax.fori_loop(0, (n_chunks - 2) // 2, body, 0)

        last_b = (n_chunks - 1) % 2
        gather_drain(last_b)
        write_start(n_chunks - 1, last_b)
        write_drain(1 - last_b)
        write_drain(last_b)

    return k(ids4d, table)


def kernel(input_ids, token_embedding):
    batch, seq = input_ids.shape
    ids4d = input_ids.T.reshape(
        seq // SUBLANES, SUBLANES, batch // IDX_MINOR, IDX_MINOR
    ).transpose(0, 2, 1, 3).astype(jnp.int32)
    rows = _sc_gather(ids4d, token_embedding)
    return rows.transpose(1, 0, 2)
